# baseline (speedup 1.0000x reference)
import functools

import jax
import jax.numpy as jnp
from jax import lax
from jax.experimental import pallas as pl
from jax.experimental.pallas import tpu as pltpu
from jax.experimental.pallas import tpu_sc as plsc

_N = 50000
_M = 16
_ORIG = 128
_NBR = 16
_AF = 64
_E = _N * _M

_NC = 2
_NS = 16
_NW = _NC * _NS

_EPS = 1e-5

_NSTR = _E // 128
_CHUNKS = 1
_CSTR = _NSTR // _CHUNKS
_CE = _CSTR * 128
_CN = _CE // _M

_BN_NODES = 200
_BN_EDGES = _BN_NODES * _M
_CTILES = _CN // _BN_NODES


def _sigmoid(x):
    return 1.0 / (1.0 + jnp.exp(-x))


def _softplus(x):
    return jnp.maximum(x, 0.0) + jnp.log1p(jnp.exp(-jnp.abs(x)))


def _sc_gather(table, idx2d, base, nstr):
    D = table.shape[1]
    kmax = -(-nstr // _NW)
    mesh = plsc.VectorSubcoreMesh(core_axis_name="c", subcore_axis_name="s")

    @functools.partial(
        pl.kernel,
        out_type=jax.ShapeDtypeStruct((nstr * 128, D), table.dtype),
        mesh=mesh,
        scratch_types=[
            pltpu.VMEM((4, 128), jnp.int32),
            pltpu.VMEM((128, D), table.dtype),
            pltpu.VMEM((128, D), table.dtype),
            pltpu.SemaphoreType.DMA,
            pltpu.SemaphoreType.DMA,
            pltpu.SemaphoreType.DMA,
            pltpu.SemaphoreType.DMA,
            pltpu.SemaphoreType.DMA,
            pltpu.SemaphoreType.DMA,
        ],
    )
    def gk(table_hbm, idx_hbm, out_hbm, idxv, rows0, rows1,
           is0, is1, is2, is3, gs0, gs1):
        w = lax.axis_index("s") * _NC + lax.axis_index("c")
        isems = (is0, is1, is2, is3)
        rows = (rows0, rows1)
        gsems = (gs0, gs1)

        def icopy(k, j):
            g = w + k * _NW

            @pl.when(g < nstr)
            def _():
                pltpu.async_copy(idx_hbm.at[base + g], idxv.at[j], isems[j])

        def iwait(k, j):
            g = w + k * _NW

            @pl.when(g < nstr)
            def _():
                pltpu.make_async_copy(
                    idx_hbm.at[base + g], idxv.at[j], isems[j]).wait()

        def gstart(k, j, s):
            g = w + k * _NW

            @pl.when(g < nstr)
            def _():
                pltpu.async_copy(table_hbm.at[idxv.at[j]], rows[s], gsems[s])

        def gwait_store(k, j, s):
            g = w + k * _NW

            @pl.when(g < nstr)
            def _():
                pltpu.make_async_copy(
                    table_hbm.at[idxv.at[j]], rows[s], gsems[s]).wait()
                pltpu.sync_copy(rows[s], out_hbm.at[pl.ds(g * 128, 128)])

        for j in range(4):
            icopy(j, j)
        iwait(0, 0)
        gstart(0, 0, 0)
        iwait(1, 1)
        gstart(1, 1, 1)

        def body(h, carry):
            k0 = 4 * h
            for j in range(4):
                k = k0 + j
                s = j % 2
                gwait_store(k, j, s)
                icopy(k + 4, j)
                iwait(k + 2, (j + 2) % 4)
                gstart(k + 2, (j + 2) % 4, s)
            return carry

        lax.fori_loop(0, -(-kmax // 4), body, 0)

    return gk(table, idx2d)


def _embed(atom_fea, mask_row, w_emb_t, wn_t):
    Bn = 2000
    grid = _N // Bn

    def body(a_ref, m_ref, w_ref, wn_ref, masked_ref, af_ref, p_ref):
        masked = a_ref[...] * m_ref[...]
        masked_ref[...] = masked
        af = jnp.dot(masked, w_ref[...], preferred_element_type=jnp.float32)
        af_ref[...] = af
        p_ref[...] = jnp.dot(af, wn_ref[...],
                             preferred_element_type=jnp.float32)

    return pl.pallas_call(
        body,
        grid=(grid,),
        in_specs=[
            pl.BlockSpec((Bn, _ORIG), lambda i: (i, 0)),
            pl.BlockSpec((1, _ORIG), lambda i: (0, 0)),
            pl.BlockSpec((_ORIG, 128), lambda i: (0, 0)),
            pl.BlockSpec((128, 128), lambda i: (0, 0)),
        ],
        out_specs=[
            pl.BlockSpec((Bn, _ORIG), lambda i: (i, 0)),
            pl.BlockSpec((Bn, 128), lambda i: (i, 0)),
            pl.BlockSpec((Bn, 128), lambda i: (i, 0)),
        ],
        out_shape=[
            jax.ShapeDtypeStruct((_N, _ORIG), jnp.float32),
            jax.ShapeDtypeStruct((_N, 128), jnp.float32),
            jax.ShapeDtypeStruct((_N, 128), jnp.float32),
        ],
    )(atom_fea, mask_row, w_emb_t, wn_t)


def _conv_stats(Z, F, af_pad, ws_t, we_t, fb_row, toff):
    def body(z_ref, f_ref, af_ref, ws_ref, we_ref, fb_ref, out_ref):
        p = jnp.dot(af_ref[...], ws_ref[...],
                    preferred_element_type=jnp.float32) + fb_ref[...]
        z = (z_ref[...].astype(jnp.float32)
             + jnp.dot(f_ref[...], we_ref[...],
                       preferred_element_type=jnp.float32))
        zsum = z.reshape(_BN_NODES, _M, 2 * _AF).sum(axis=1)
        s1 = z.sum(axis=0, keepdims=True)
        s2 = (z * z).sum(axis=0, keepdims=True)
        t1 = (p * zsum).sum(axis=0, keepdims=True)
        p1 = p.sum(axis=0, keepdims=True)
        p2 = (p * p).sum(axis=0, keepdims=True)
        blk = jnp.concatenate([s1, s2, t1, p1, p2,
                               jnp.zeros((3, 2 * _AF), jnp.float32)], axis=0)

        @pl.when(pl.program_id(0) == 0)
        def _():
            out_ref[...] = blk

        @pl.when(pl.program_id(0) != 0)
        def _():
            out_ref[...] += blk

    return pl.pallas_call(
        body,
        grid=(_CTILES,),
        in_specs=[
            pl.BlockSpec((_BN_EDGES, 2 * _AF), lambda i: (i, 0)),
            pl.BlockSpec((_BN_EDGES, _NBR), lambda i: (i + toff, 0)),
            pl.BlockSpec((_BN_NODES, 128), lambda i: (i + toff, 0)),
            pl.BlockSpec((128, 2 * _AF), lambda i: (0, 0)),
            pl.BlockSpec((_NBR, 2 * _AF), lambda i: (0, 0)),
            pl.BlockSpec((1, 2 * _AF), lambda i: (0, 0)),
        ],
        out_specs=pl.BlockSpec((8, 2 * _AF), lambda i: (0, 0)),
        out_shape=jax.ShapeDtypeStruct((8, 2 * _AF), jnp.float32),
    )(Z, F, af_pad, ws_t, we_t, fb_row)


def _conv_apply(Z, F, af_pad, ws_t, we_t, fb_row, sc1, sh1, toff):
    def body(z_ref, f_ref, af_ref, ws_ref, we_ref, fb_ref,
             sc_ref, sh_ref, s_ref, q_ref):
        p = jnp.dot(af_ref[...], ws_ref[...],
                    preferred_element_type=jnp.float32) + fb_ref[...]
        z = (z_ref[...].astype(jnp.float32)
             + jnp.dot(f_ref[...], we_ref[...],
                       preferred_element_type=jnp.float32))
        gated = z.reshape(_BN_NODES, _M, 2 * _AF) + p[:, None, :]
        gated = gated * sc_ref[...][None, :, :] + sh_ref[...][None, :, :]
        filt = gated[:, :, :_AF]
        core = gated[:, :, _AF:]
        y = _sigmoid(filt) * _softplus(core)
        s = y.sum(axis=1)
        s_pad = jnp.concatenate(
            [s, jnp.zeros((_BN_NODES, _AF), jnp.float32)], axis=1)
        s_ref[...] = s_pad
        q1 = s.sum(axis=0, keepdims=True)
        q2 = (s * s).sum(axis=0, keepdims=True)
        blk = jnp.concatenate([q1, q2,
                               jnp.zeros((6, _AF), jnp.float32)], axis=0)

        @pl.when(pl.program_id(0) == 0)
        def _():
            q_ref[...] = blk

        @pl.when(pl.program_id(0) != 0)
        def _():
            q_ref[...] += blk

    return pl.pallas_call(
        body,
        grid=(_CTILES,),
        in_specs=[
            pl.BlockSpec((_BN_EDGES, 2 * _AF), lambda i: (i, 0)),
            pl.BlockSpec((_BN_EDGES, _NBR), lambda i: (i + toff, 0)),
            pl.BlockSpec((_BN_NODES, 128), lambda i: (i + toff, 0)),
            pl.BlockSpec((128, 2 * _AF), lambda i: (0, 0)),
            pl.BlockSpec((_NBR, 2 * _AF), lambda i: (0, 0)),
            pl.BlockSpec((1, 2 * _AF), lambda i: (0, 0)),
            pl.BlockSpec((1, 2 * _AF), lambda i: (0, 0)),
            pl.BlockSpec((1, 2 * _AF), lambda i: (0, 0)),
        ],
        out_specs=[
            pl.BlockSpec((_BN_NODES, 128), lambda i: (i, 0)),
            pl.BlockSpec((8, _AF), lambda i: (0, 0)),
        ],
        out_shape=[
            jax.ShapeDtypeStruct((_CN, 128), jnp.float32),
            jax.ShapeDtypeStruct((8, _AF), jnp.float32),
        ],
    )(Z, F, af_pad, ws_t, we_t, fb_row, sc1, sh1)


def _conv_finalize(af_pad, s_pad, sc2, sh2, lanemask, wnext_t=None):
    Bn = 2000
    grid = _N // Bn

    if wnext_t is None:
        def body(af_ref, s_ref, sc_ref, sh_ref, lm_ref, out_ref):
            out_ref[...] = _softplus(
                af_ref[...] + s_ref[...] * sc_ref[...] + sh_ref[...]
            ) * lm_ref[...]

        return pl.pallas_call(
            body,
            grid=(grid,),
            in_specs=[
                pl.BlockSpec((Bn, 128), lambda i: (i, 0)),
                pl.BlockSpec((Bn, 128), lambda i: (i, 0)),
                pl.BlockSpec((1, 128), lambda i: (0, 0)),
                pl.BlockSpec((1, 128), lambda i: (0, 0)),
                pl.BlockSpec((1, 128), lambda i: (0, 0)),
            ],
            out_specs=pl.BlockSpec((Bn, 128), lambda i: (i, 0)),
            out_shape=jax.ShapeDtypeStruct((_N, 128), jnp.float32),
        )(af_pad, s_pad, sc2, sh2, lanemask)

    def body(af_ref, s_ref, sc_ref, sh_ref, lm_ref, wn_ref, out_ref, p_ref):
        af_new = _softplus(
            af_ref[...] + s_ref[...] * sc_ref[...] + sh_ref[...]) * lm_ref[...]
        out_ref[...] = af_new
        p_ref[...] = jnp.dot(af_new, wn_ref[...],
                             preferred_element_type=jnp.float32)

    return pl.pallas_call(
        body,
        grid=(grid,),
        in_specs=[
            pl.BlockSpec((Bn, 128), lambda i: (i, 0)),
            pl.BlockSpec((Bn, 128), lambda i: (i, 0)),
            pl.BlockSpec((1, 128), lambda i: (0, 0)),
            pl.BlockSpec((1, 128), lambda i: (0, 0)),
            pl.BlockSpec((1, 128), lambda i: (0, 0)),
            pl.BlockSpec((128, 128), lambda i: (0, 0)),
        ],
        out_specs=[
            pl.BlockSpec((Bn, 128), lambda i: (i, 0)),
            pl.BlockSpec((Bn, 128), lambda i: (i, 0)),
        ],
        out_shape=[
            jax.ShapeDtypeStruct((_N, 128), jnp.float32),
            jax.ShapeDtypeStruct((_N, 128), jnp.float32),
        ],
    )(af_pad, s_pad, sc2, sh2, lanemask, wnext_t)


def _readout(rows, fc1_wt, fc1_b, fc2_wt, fc2_b, out_wt, out_b, ncry, apc):
    tot = ncry * apc

    def body(r_ref, w1_ref, b1_ref, w2_ref, b2_ref, wo_ref, bo_ref, o_ref):
        r = r_ref[...]
        nrm = jnp.sqrt((r * r).sum(axis=1, keepdims=True))
        g = r / jnp.maximum(nrm, 1e-12)
        pooled = g.reshape(ncry, apc, 128).mean(axis=1)
        h = _softplus(jnp.dot(pooled, w1_ref[...],
                              preferred_element_type=jnp.float32) + b1_ref[...])
        h = _softplus(jnp.dot(h, w2_ref[...],
                              preferred_element_type=jnp.float32) + b2_ref[...])
        props = (jnp.dot(h, wo_ref[...], preferred_element_type=jnp.float32)
                 + bo_ref[...])
        o_ref[...] = props

    return pl.pallas_call(
        body,
        grid=(1,),
        in_specs=[
            pl.BlockSpec((tot, 128), lambda i: (0, 0)),
            pl.BlockSpec((128, _AF), lambda i: (0, 0)),
            pl.BlockSpec((1, _AF), lambda i: (0, 0)),
            pl.BlockSpec((_AF, _AF), lambda i: (0, 0)),
            pl.BlockSpec((1, _AF), lambda i: (0, 0)),
            pl.BlockSpec((_AF, 1), lambda i: (0, 0)),
            pl.BlockSpec((1, 1), lambda i: (0, 0)),
        ],
        out_specs=pl.BlockSpec((ncry, 1), lambda i: (0, 0)),
        out_shape=jax.ShapeDtypeStruct((ncry, 1), jnp.float32),
    )(rows, fc1_wt, fc1_b, fc2_wt, fc2_b, out_wt, out_b)


def kernel(atom_fea, nbr_fea, nbr_fea_idx, crystal_atom_idx, mask, w_emb,
           conv0_fc_w, conv0_fc_b, conv0_bn1_g, conv0_bn1_b, conv0_bn2_g,
           conv0_bn2_b, conv1_fc_w, conv1_fc_b, conv1_bn1_g, conv1_bn1_b,
           conv1_bn2_g, conv1_bn2_b, conv2_fc_w, conv2_fc_b, conv2_bn1_g,
           conv2_bn1_b, conv2_bn2_g, conv2_bn2_b, fc1_w, fc1_b, fc2_w, fc2_b,
           out_w, out_b):
    f32 = jnp.float32
    zpad64 = jnp.zeros((_AF, 2 * _AF), f32)

    def _wsplit(fw):
        fwt = fw.T
        ws_t = jnp.concatenate([fwt[:_AF], zpad64], axis=0)
        wn_t = jnp.concatenate([fwt[_AF:2 * _AF], zpad64], axis=0)
        we_t = fwt[2 * _AF:]
        return ws_t, wn_t, we_t

    wsplits = [_wsplit(conv0_fc_w), _wsplit(conv1_fc_w), _wsplit(conv2_fc_w)]

    mask_row = mask.reshape(1, _ORIG)
    w_emb_t = jnp.concatenate(
        [w_emb.T, jnp.zeros((_ORIG, 128 - _AF), f32)], axis=1)
    masked, af, P_nbr = _embed(atom_fea, mask_row, w_emb_t, wsplits[0][1])

    idx2d = nbr_fea_idx.reshape(_NSTR, 128).astype(jnp.int32)
    F = nbr_fea.reshape(_E, _NBR)
    lanemask = jnp.concatenate(
        [jnp.ones((1, _AF), f32), jnp.zeros((1, _AF), f32)], axis=1)

    convp = [(conv0_fc_b, conv0_bn1_g, conv0_bn1_b, conv0_bn2_g, conv0_bn2_b),
             (conv1_fc_b, conv1_bn1_g, conv1_bn1_b, conv1_bn2_g, conv1_bn2_b),
             (conv2_fc_b, conv2_bn1_g, conv2_bn1_b, conv2_bn2_g, conv2_bn2_b)]

    nm = float(_E)
    for li, (fb, g1, b1, g2, b2) in enumerate(convp):
        ws_t, _, we_t = wsplits[li]
        fb_row = fb.reshape(1, 2 * _AF)

        Zc = [_sc_gather(P_nbr, idx2d, c * _CSTR, _CSTR)
              for c in range(_CHUNKS)]
        stats_c = [_conv_stats(Zc[c], F, af, ws_t, we_t, fb_row, c * _CTILES)
                   for c in range(_CHUNKS)]
        stats = stats_c[0]
        for c in range(1, _CHUNKS):
            stats = stats + stats_c[c]

        s1, s2, t1 = stats[0], stats[1], stats[2]
        p1, p2 = stats[3], stats[4]
        colsum = s1 + _M * p1
        colsq = s2 + 2.0 * t1 + _M * p2
        mu = colsum / nm
        var = colsq / nm - mu * mu
        inv = g1 / jnp.sqrt(var + _EPS)
        sc1 = inv.reshape(1, 2 * _AF)
        sh1 = (b1 - mu * inv).reshape(1, 2 * _AF)

        sq = [_conv_apply(Zc[c], F, af, ws_t, we_t, fb_row, sc1, sh1,
                          c * _CTILES)
              for c in range(_CHUNKS)]
        s_pad = jnp.concatenate([x[0] for x in sq], axis=0)
        q = sq[0][1]
        for c in range(1, _CHUNKS):
            q = q + sq[c][1]

        mu2 = q[0] / float(_N)
        var2 = q[1] / float(_N) - mu2 * mu2
        inv2 = g2 / jnp.sqrt(var2 + _EPS)
        sc2 = jnp.concatenate([inv2, jnp.zeros((_AF,), f32)]).reshape(1, 128)
        sh2 = jnp.concatenate([b2 - mu2 * inv2,
                               jnp.zeros((_AF,), f32)]).reshape(1, 128)

        if li < 2:
            af, P_nbr = _conv_finalize(af, s_pad, sc2, sh2, lanemask,
                                       wsplits[li + 1][1])
        else:
            af = _conv_finalize(af, s_pad, sc2, sh2, lanemask)

    ncry, apc = crystal_atom_idx.shape
    cidx = crystal_atom_idx.reshape((ncry * apc) // 128, 128).astype(jnp.int32)
    rows = _sc_gather(af, cidx, 0, (ncry * apc) // 128)

    fc1_wt = jnp.concatenate([fc1_w.T, jnp.zeros((_AF, _AF), f32)], axis=0)
    props = _readout(rows, fc1_wt, fc1_b.reshape(1, _AF), fc2_w.T,
                     fc2_b.reshape(1, _AF), out_w.T, out_b.reshape(1, 1),
                     ncry, apc)
    return props, masked


# speedup vs baseline: 1.1200x; 1.1200x over previous
import functools

import jax
import jax.numpy as jnp
from jax import lax
from jax.experimental import pallas as pl
from jax.experimental.pallas import tpu as pltpu
from jax.experimental.pallas import tpu_sc as plsc

_N = 50000
_M = 16
_ORIG = 128
_NBR = 16
_AF = 64
_E = _N * _M

_NC = 2
_NS = 16
_NW = _NC * _NS

_EPS = 1e-5

_NSTR = _E // 128
_CHUNKS = 1
_CSTR = _NSTR // _CHUNKS
_CE = _CSTR * 128
_CN = _CE // _M

_BN_NODES = 400
_BN_EDGES = _BN_NODES * _M
_CTILES = _CN // _BN_NODES


def _sigmoid(x):
    return 1.0 / (1.0 + jnp.exp(-x))


def _softplus(x):
    return jnp.maximum(x, 0.0) + jnp.log1p(jnp.exp(-jnp.abs(x)))


def _sc_gather(table, idx2d, base, nstr):
    D = table.shape[1]
    kmax = -(-nstr // _NW)
    mesh = plsc.VectorSubcoreMesh(core_axis_name="c", subcore_axis_name="s")

    @functools.partial(
        pl.kernel,
        out_type=jax.ShapeDtypeStruct((nstr * 128, D), table.dtype),
        mesh=mesh,
        scratch_types=[
            pltpu.VMEM((4, 128), jnp.int32),
            pltpu.VMEM((128, D), table.dtype),
            pltpu.VMEM((128, D), table.dtype),
            pltpu.SemaphoreType.DMA,
            pltpu.SemaphoreType.DMA,
            pltpu.SemaphoreType.DMA,
            pltpu.SemaphoreType.DMA,
            pltpu.SemaphoreType.DMA,
            pltpu.SemaphoreType.DMA,
        ],
    )
    def gk(table_hbm, idx_hbm, out_hbm, idxv, rows0, rows1,
           is0, is1, is2, is3, gs0, gs1):
        w = lax.axis_index("s") * _NC + lax.axis_index("c")
        isems = (is0, is1, is2, is3)
        rows = (rows0, rows1)
        gsems = (gs0, gs1)

        def icopy(k, j):
            g = w + k * _NW

            @pl.when(g < nstr)
            def _():
                pltpu.async_copy(idx_hbm.at[base + g], idxv.at[j], isems[j])

        def iwait(k, j):
            g = w + k * _NW

            @pl.when(g < nstr)
            def _():
                pltpu.make_async_copy(
                    idx_hbm.at[base + g], idxv.at[j], isems[j]).wait()

        def gstart(k, j, s):
            g = w + k * _NW

            @pl.when(g < nstr)
            def _():
                pltpu.async_copy(table_hbm.at[idxv.at[j]], rows[s], gsems[s])

        def gwait_store(k, j, s):
            g = w + k * _NW

            @pl.when(g < nstr)
            def _():
                pltpu.make_async_copy(
                    table_hbm.at[idxv.at[j]], rows[s], gsems[s]).wait()
                pltpu.sync_copy(rows[s], out_hbm.at[pl.ds(g * 128, 128)])

        for j in range(4):
            icopy(j, j)
        iwait(0, 0)
        gstart(0, 0, 0)
        iwait(1, 1)
        gstart(1, 1, 1)

        def body(h, carry):
            k0 = 4 * h
            for j in range(4):
                k = k0 + j
                s = j % 2
                gwait_store(k, j, s)
                icopy(k + 4, j)
                iwait(k + 2, (j + 2) % 4)
                gstart(k + 2, (j + 2) % 4, s)
            return carry

        lax.fori_loop(0, -(-kmax // 4), body, 0)

    return gk(table, idx2d)


def _embed(atom_fea, mask_row, w_emb_t, wn_t):
    Bn = 2000
    grid = _N // Bn

    def body(a_ref, m_ref, w_ref, wn_ref, masked_ref, af_ref, p_ref):
        masked = a_ref[...] * m_ref[...]
        masked_ref[...] = masked
        af = jnp.dot(masked, w_ref[...], preferred_element_type=jnp.float32)
        af_ref[...] = af
        p_ref[...] = jnp.dot(af, wn_ref[...],
                             preferred_element_type=jnp.float32)

    return pl.pallas_call(
        body,
        grid=(grid,),
        in_specs=[
            pl.BlockSpec((Bn, _ORIG), lambda i: (i, 0)),
            pl.BlockSpec((1, _ORIG), lambda i: (0, 0)),
            pl.BlockSpec((_ORIG, 128), lambda i: (0, 0)),
            pl.BlockSpec((128, 128), lambda i: (0, 0)),
        ],
        out_specs=[
            pl.BlockSpec((Bn, _ORIG), lambda i: (i, 0)),
            pl.BlockSpec((Bn, 128), lambda i: (i, 0)),
            pl.BlockSpec((Bn, 128), lambda i: (i, 0)),
        ],
        out_shape=[
            jax.ShapeDtypeStruct((_N, _ORIG), jnp.float32),
            jax.ShapeDtypeStruct((_N, 128), jnp.float32),
            jax.ShapeDtypeStruct((_N, 128), jnp.float32),
        ],
    )(atom_fea, mask_row, w_emb_t, wn_t)


def _conv_stats(Z, F, af_pad, ws_t, we_t, fb_row, g1_row, b1_row, toff):
    nm = float(_E)

    def body(z_ref, f_ref, af_ref, ws_ref, we_ref, fb_ref, g1_ref, b1_ref,
             out_ref):
        p = jnp.dot(af_ref[...], ws_ref[...],
                    preferred_element_type=jnp.float32) + fb_ref[...]
        z = (z_ref[...].astype(jnp.float32)
             + jnp.dot(f_ref[...], we_ref[...],
                       preferred_element_type=jnp.float32))
        zsum = z.reshape(_BN_NODES, _M, 2 * _AF).sum(axis=1)
        s1 = z.sum(axis=0, keepdims=True)
        s2 = (z * z).sum(axis=0, keepdims=True)
        t1 = (p * zsum).sum(axis=0, keepdims=True)
        p1 = p.sum(axis=0, keepdims=True)
        p2 = (p * p).sum(axis=0, keepdims=True)
        blk = jnp.concatenate([s1, s2, t1, p1, p2,
                               jnp.zeros((3, 2 * _AF), jnp.float32)], axis=0)

        @pl.when(pl.program_id(0) == 0)
        def _():
            out_ref[0] = blk

        @pl.when(pl.program_id(0) != 0)
        def _():
            out_ref[0] += blk

        @pl.when(pl.program_id(0) == _CTILES - 1)
        def _():
            acc = out_ref[0]
            colsum = acc[0:1] + float(_M) * acc[3:4]
            colsq = acc[1:2] + 2.0 * acc[2:3] + float(_M) * acc[4:5]
            mu = colsum / nm
            var = colsq / nm - mu * mu
            inv = g1_ref[...] * jax.lax.rsqrt(var + _EPS)
            sh = b1_ref[...] - mu * inv
            out_ref[1] = jnp.concatenate(
                [inv, sh, jnp.zeros((6, 2 * _AF), jnp.float32)], axis=0)

    return pl.pallas_call(
        body,
        grid=(_CTILES,),
        in_specs=[
            pl.BlockSpec((_BN_EDGES, 2 * _AF), lambda i: (i, 0)),
            pl.BlockSpec((_BN_EDGES, _NBR), lambda i: (i + toff, 0)),
            pl.BlockSpec((_BN_NODES, 128), lambda i: (i + toff, 0)),
            pl.BlockSpec((128, 2 * _AF), lambda i: (0, 0)),
            pl.BlockSpec((_NBR, 2 * _AF), lambda i: (0, 0)),
            pl.BlockSpec((1, 2 * _AF), lambda i: (0, 0)),
            pl.BlockSpec((1, 2 * _AF), lambda i: (0, 0)),
            pl.BlockSpec((1, 2 * _AF), lambda i: (0, 0)),
        ],
        out_specs=pl.BlockSpec((2, 8, 2 * _AF), lambda i: (0, 0, 0)),
        out_shape=jax.ShapeDtypeStruct((2, 8, 2 * _AF), jnp.float32),
    )(Z, F, af_pad, ws_t, we_t, fb_row, g1_row, b1_row)


def _conv_apply(Z, F, af_pad, ws_t, we_t, fb_row, st, g2_row, b2_row, toff):
    nn = float(_N)

    def body(z_ref, f_ref, af_ref, ws_ref, we_ref, fb_ref, st_ref,
             g2_ref, b2_ref, s_ref, q_ref):
        p = jnp.dot(af_ref[...], ws_ref[...],
                    preferred_element_type=jnp.float32) + fb_ref[...]
        z = (z_ref[...].astype(jnp.float32)
             + jnp.dot(f_ref[...], we_ref[...],
                       preferred_element_type=jnp.float32))
        sc1 = st_ref[0, 0:1, :]
        sh1 = st_ref[0, 1:2, :]
        gated = z.reshape(_BN_NODES, _M, 2 * _AF) + p[:, None, :]
        gated = gated * sc1[None, :, :] + sh1[None, :, :]
        filt = gated[:, :, :_AF]
        core = gated[:, :, _AF:]
        y = _sigmoid(filt) * _softplus(core)
        s = y.sum(axis=1)
        s_pad = jnp.concatenate(
            [s, jnp.zeros((_BN_NODES, _AF), jnp.float32)], axis=1)
        s_ref[...] = s_pad
        q1 = (s_pad.sum(axis=0, keepdims=True))
        q2 = (s_pad * s_pad).sum(axis=0, keepdims=True)
        blk = jnp.concatenate([q1, q2], axis=0)

        @pl.when(pl.program_id(0) == 0)
        def _():
            q_ref[0] = blk

        @pl.when(pl.program_id(0) != 0)
        def _():
            q_ref[0] += blk

        @pl.when(pl.program_id(0) == _CTILES - 1)
        def _():
            acc = q_ref[0]
            mu2 = acc[0:1] / nn
            var2 = acc[1:2] / nn - mu2 * mu2
            inv2 = g2_ref[...] * jax.lax.rsqrt(var2 + _EPS)
            sh2 = b2_ref[...] - mu2 * inv2
            q_ref[1] = jnp.concatenate([inv2, sh2], axis=0)

    return pl.pallas_call(
        body,
        grid=(_CTILES,),
        in_specs=[
            pl.BlockSpec((_BN_EDGES, 2 * _AF), lambda i: (i, 0)),
            pl.BlockSpec((_BN_EDGES, _NBR), lambda i: (i + toff, 0)),
            pl.BlockSpec((_BN_NODES, 128), lambda i: (i + toff, 0)),
            pl.BlockSpec((128, 2 * _AF), lambda i: (0, 0)),
            pl.BlockSpec((_NBR, 2 * _AF), lambda i: (0, 0)),
            pl.BlockSpec((1, 2 * _AF), lambda i: (0, 0)),
            pl.BlockSpec((1, 8, 2 * _AF), lambda i: (1, 0, 0)),
            pl.BlockSpec((1, 2 * _AF), lambda i: (0, 0)),
            pl.BlockSpec((1, 2 * _AF), lambda i: (0, 0)),
        ],
        out_specs=[
            pl.BlockSpec((_BN_NODES, 128), lambda i: (i, 0)),
            pl.BlockSpec((2, 2, 2 * _AF), lambda i: (0, 0, 0)),
        ],
        out_shape=[
            jax.ShapeDtypeStruct((_CN, 128), jnp.float32),
            jax.ShapeDtypeStruct((2, 2, 2 * _AF), jnp.float32),
        ],
    )(Z, F, af_pad, ws_t, we_t, fb_row, st, g2_row, b2_row)


def _conv_finalize(af_pad, s_pad, q, lanemask, wnext_t=None):
    Bn = 2000
    grid = _N // Bn

    if wnext_t is None:
        def body(af_ref, s_ref, q_ref, lm_ref, out_ref):
            sc2 = q_ref[0, 0:1, :]
            sh2 = q_ref[0, 1:2, :]
            out_ref[...] = _softplus(
                af_ref[...] + s_ref[...] * sc2 + sh2) * lm_ref[...]

        return pl.pallas_call(
            body,
            grid=(grid,),
            in_specs=[
                pl.BlockSpec((Bn, 128), lambda i: (i, 0)),
                pl.BlockSpec((Bn, 128), lambda i: (i, 0)),
                pl.BlockSpec((1, 2, 128), lambda i: (1, 0, 0)),
                pl.BlockSpec((1, 128), lambda i: (0, 0)),
            ],
            out_specs=pl.BlockSpec((Bn, 128), lambda i: (i, 0)),
            out_shape=jax.ShapeDtypeStruct((_N, 128), jnp.float32),
        )(af_pad, s_pad, q, lanemask)

    def body(af_ref, s_ref, q_ref, lm_ref, wn_ref, out_ref, p_ref):
        sc2 = q_ref[0, 0:1, :]
        sh2 = q_ref[0, 1:2, :]
        af_new = _softplus(
            af_ref[...] + s_ref[...] * sc2 + sh2) * lm_ref[...]
        out_ref[...] = af_new
        p_ref[...] = jnp.dot(af_new, wn_ref[...],
                             preferred_element_type=jnp.float32)

    return pl.pallas_call(
        body,
        grid=(grid,),
        in_specs=[
            pl.BlockSpec((Bn, 128), lambda i: (i, 0)),
            pl.BlockSpec((Bn, 128), lambda i: (i, 0)),
            pl.BlockSpec((1, 2, 128), lambda i: (1, 0, 0)),
            pl.BlockSpec((1, 128), lambda i: (0, 0)),
            pl.BlockSpec((128, 128), lambda i: (0, 0)),
        ],
        out_specs=[
            pl.BlockSpec((Bn, 128), lambda i: (i, 0)),
            pl.BlockSpec((Bn, 128), lambda i: (i, 0)),
        ],
        out_shape=[
            jax.ShapeDtypeStruct((_N, 128), jnp.float32),
            jax.ShapeDtypeStruct((_N, 128), jnp.float32),
        ],
    )(af_pad, s_pad, q, lanemask, wnext_t)


def _readout(rows, fc1_wt, fc1_b, fc2_wt, fc2_b, out_wt, out_b, ncry, apc):
    tot = ncry * apc

    def body(r_ref, w1_ref, b1_ref, w2_ref, b2_ref, wo_ref, bo_ref, o_ref):
        r = r_ref[...]
        nrm = jnp.sqrt((r * r).sum(axis=1, keepdims=True))
        g = r / jnp.maximum(nrm, 1e-12)
        pooled = g.reshape(ncry, apc, 128).mean(axis=1)
        h = _softplus(jnp.dot(pooled, w1_ref[...],
                              preferred_element_type=jnp.float32) + b1_ref[...])
        h = _softplus(jnp.dot(h, w2_ref[...],
                              preferred_element_type=jnp.float32) + b2_ref[...])
        props = (jnp.dot(h, wo_ref[...], preferred_element_type=jnp.float32)
                 + bo_ref[...])
        o_ref[...] = props

    return pl.pallas_call(
        body,
        grid=(1,),
        in_specs=[
            pl.BlockSpec((tot, 128), lambda i: (0, 0)),
            pl.BlockSpec((128, _AF), lambda i: (0, 0)),
            pl.BlockSpec((1, _AF), lambda i: (0, 0)),
            pl.BlockSpec((_AF, _AF), lambda i: (0, 0)),
            pl.BlockSpec((1, _AF), lambda i: (0, 0)),
            pl.BlockSpec((_AF, 1), lambda i: (0, 0)),
            pl.BlockSpec((1, 1), lambda i: (0, 0)),
        ],
        out_specs=pl.BlockSpec((ncry, 1), lambda i: (0, 0)),
        out_shape=jax.ShapeDtypeStruct((ncry, 1), jnp.float32),
    )(rows, fc1_wt, fc1_b, fc2_wt, fc2_b, out_wt, out_b)


def kernel(atom_fea, nbr_fea, nbr_fea_idx, crystal_atom_idx, mask, w_emb,
           conv0_fc_w, conv0_fc_b, conv0_bn1_g, conv0_bn1_b, conv0_bn2_g,
           conv0_bn2_b, conv1_fc_w, conv1_fc_b, conv1_bn1_g, conv1_bn1_b,
           conv1_bn2_g, conv1_bn2_b, conv2_fc_w, conv2_fc_b, conv2_bn1_g,
           conv2_bn1_b, conv2_bn2_g, conv2_bn2_b, fc1_w, fc1_b, fc2_w, fc2_b,
           out_w, out_b):
    f32 = jnp.float32
    zpad64 = jnp.zeros((_AF, 2 * _AF), f32)

    def _wsplit(fw):
        fwt = fw.T
        ws_t = jnp.concatenate([fwt[:_AF], zpad64], axis=0)
        wn_t = jnp.concatenate([fwt[_AF:2 * _AF], zpad64], axis=0)
        we_t = fwt[2 * _AF:]
        return ws_t, wn_t, we_t

    wsplits = [_wsplit(conv0_fc_w), _wsplit(conv1_fc_w), _wsplit(conv2_fc_w)]

    mask_row = mask.reshape(1, _ORIG)
    w_emb_t = jnp.concatenate(
        [w_emb.T, jnp.zeros((_ORIG, 128 - _AF), f32)], axis=1)
    masked, af, P_nbr = _embed(atom_fea, mask_row, w_emb_t, wsplits[0][1])

    idx2d = nbr_fea_idx.reshape(_NSTR, 128).astype(jnp.int32)
    F = nbr_fea.reshape(_E, _NBR)
    lanemask = jnp.concatenate(
        [jnp.ones((1, _AF), f32), jnp.zeros((1, _AF), f32)], axis=1)

    convp = [(conv0_fc_b, conv0_bn1_g, conv0_bn1_b, conv0_bn2_g, conv0_bn2_b),
             (conv1_fc_b, conv1_bn1_g, conv1_bn1_b, conv1_bn2_g, conv1_bn2_b),
             (conv2_fc_b, conv2_bn1_g, conv2_bn1_b, conv2_bn2_g, conv2_bn2_b)]

    zrow = jnp.zeros((1, _AF), f32)
    for li, (fb, g1, b1, g2, b2) in enumerate(convp):
        ws_t, _, we_t = wsplits[li]
        fb_row = fb.reshape(1, 2 * _AF)
        g1_row = g1.reshape(1, 2 * _AF)
        b1_row = b1.reshape(1, 2 * _AF)
        g2_row = jnp.concatenate([g2.reshape(1, _AF), zrow], axis=1)
        b2_row = jnp.concatenate([b2.reshape(1, _AF), zrow], axis=1)

        Z = _sc_gather(P_nbr, idx2d, 0, _NSTR)
        st = _conv_stats(Z, F, af, ws_t, we_t, fb_row, g1_row, b1_row, 0)
        s_pad, q = _conv_apply(Z, F, af, ws_t, we_t, fb_row, st,
                               g2_row, b2_row, 0)

        if li < 2:
            af, P_nbr = _conv_finalize(af, s_pad, q, lanemask,
                                       wsplits[li + 1][1])
        else:
            af = _conv_finalize(af, s_pad, q, lanemask)

    ncry, apc = crystal_atom_idx.shape
    cidx = crystal_atom_idx.reshape((ncry * apc) // 128, 128).astype(jnp.int32)
    rows = _sc_gather(af, cidx, 0, (ncry * apc) // 128)

    fc1_wt = jnp.concatenate([fc1_w.T, jnp.zeros((_AF, _AF), f32)], axis=0)
    props = _readout(rows, fc1_wt, fc1_b.reshape(1, _AF), fc2_w.T,
                     fc2_b.reshape(1, _AF), out_w.T, out_b.reshape(1, 1),
                     ncry, apc)
    return props, masked


# speedup vs baseline: 1.1259x; 1.0052x over previous
import functools

import jax
import jax.numpy as jnp
from jax import lax
from jax.experimental import pallas as pl
from jax.experimental.pallas import tpu as pltpu
from jax.experimental.pallas import tpu_sc as plsc

_N = 50000
_M = 16
_ORIG = 128
_NBR = 16
_AF = 64
_E = _N * _M

_NC = 2
_NS = 16
_NW = _NC * _NS

_EPS = 1e-5

_NSTR = _E // 128
_CHUNKS = 1
_CSTR = _NSTR // _CHUNKS
_CE = _CSTR * 128
_CN = _CE // _M

_BN_NODES = 400
_BN_EDGES = _BN_NODES * _M
_CTILES = _CN // _BN_NODES


def _sigmoid(x):
    return 1.0 / (1.0 + jnp.exp(-x))


def _softplus(x):
    return jnp.maximum(x, 0.0) + jnp.log1p(jnp.exp(-jnp.abs(x)))


def _sc_gather(table, idx2d, base, nstr):
    D = table.shape[1]
    kmax = -(-nstr // _NW)
    mesh = plsc.VectorSubcoreMesh(core_axis_name="c", subcore_axis_name="s")

    @functools.partial(
        pl.kernel,
        out_type=jax.ShapeDtypeStruct((nstr * 128, D), table.dtype),
        mesh=mesh,
        scratch_types=[
            pltpu.VMEM((8, 128), jnp.int32),
            pltpu.VMEM((128, D), table.dtype),
            pltpu.VMEM((128, D), table.dtype),
            pltpu.VMEM((128, D), table.dtype),
            pltpu.VMEM((128, D), table.dtype),
            pltpu.SemaphoreType.DMA,
            pltpu.SemaphoreType.DMA,
            pltpu.SemaphoreType.DMA,
            pltpu.SemaphoreType.DMA,
            pltpu.SemaphoreType.DMA,
            pltpu.SemaphoreType.DMA,
            pltpu.SemaphoreType.DMA,
            pltpu.SemaphoreType.DMA,
            pltpu.SemaphoreType.DMA,
            pltpu.SemaphoreType.DMA,
            pltpu.SemaphoreType.DMA,
            pltpu.SemaphoreType.DMA,
        ],
    )
    def gk(table_hbm, idx_hbm, out_hbm, idxv, rows0, rows1, rows2, rows3,
           is0, is1, is2, is3, is4, is5, is6, is7,
           gs0, gs1, gs2, gs3):
        w = lax.axis_index("s") * _NC + lax.axis_index("c")
        isems = (is0, is1, is2, is3, is4, is5, is6, is7)
        rows = (rows0, rows1, rows2, rows3)
        gsems = (gs0, gs1, gs2, gs3)

        def icopy(k, j):
            g = w + k * _NW

            @pl.when(g < nstr)
            def _():
                pltpu.async_copy(idx_hbm.at[base + g], idxv.at[j], isems[j])

        def iwait(k, j):
            g = w + k * _NW

            @pl.when(g < nstr)
            def _():
                pltpu.make_async_copy(
                    idx_hbm.at[base + g], idxv.at[j], isems[j]).wait()

        def gstart(k, j, s):
            g = w + k * _NW

            @pl.when(g < nstr)
            def _():
                pltpu.async_copy(table_hbm.at[idxv.at[j]], rows[s], gsems[s])

        def gwait_store(k, j, s):
            g = w + k * _NW

            @pl.when(g < nstr)
            def _():
                pltpu.make_async_copy(
                    table_hbm.at[idxv.at[j]], rows[s], gsems[s]).wait()
                pltpu.sync_copy(rows[s], out_hbm.at[pl.ds(g * 128, 128)])

        for j in range(8):
            icopy(j, j)
        for j in range(4):
            iwait(j, j)
            gstart(j, j, j)

        def body(h, carry):
            k0 = 8 * h
            for j in range(8):
                k = k0 + j
                s = j % 4
                gwait_store(k, j, s)
                icopy(k + 8, j)
                iwait(k + 4, (j + 4) % 8)
                gstart(k + 4, (j + 4) % 8, s)
            return carry

        lax.fori_loop(0, -(-kmax // 8), body, 0)

    return gk(table, idx2d)


def _embed(atom_fea, mask_row, w_emb_t, wn_t):
    Bn = 2000
    grid = _N // Bn

    def body(a_ref, m_ref, w_ref, wn_ref, masked_ref, af_ref, p_ref):
        masked = a_ref[...] * m_ref[...]
        masked_ref[...] = masked
        af = jnp.dot(masked, w_ref[...], preferred_element_type=jnp.float32)
        af_ref[...] = af
        p_ref[...] = jnp.dot(af, wn_ref[...],
                             preferred_element_type=jnp.float32)

    return pl.pallas_call(
        body,
        grid=(grid,),
        in_specs=[
            pl.BlockSpec((Bn, _ORIG), lambda i: (i, 0)),
            pl.BlockSpec((1, _ORIG), lambda i: (0, 0)),
            pl.BlockSpec((_ORIG, 128), lambda i: (0, 0)),
            pl.BlockSpec((128, 128), lambda i: (0, 0)),
        ],
        out_specs=[
            pl.BlockSpec((Bn, _ORIG), lambda i: (i, 0)),
            pl.BlockSpec((Bn, 128), lambda i: (i, 0)),
            pl.BlockSpec((Bn, 128), lambda i: (i, 0)),
        ],
        out_shape=[
            jax.ShapeDtypeStruct((_N, _ORIG), jnp.float32),
            jax.ShapeDtypeStruct((_N, 128), jnp.float32),
            jax.ShapeDtypeStruct((_N, 128), jnp.float32),
        ],
    )(atom_fea, mask_row, w_emb_t, wn_t)


def _conv_stats(Z, F, af_pad, ws_t, we_t, fb_row, g1_row, b1_row, toff):
    nm = float(_E)

    def body(z_ref, f_ref, af_ref, ws_ref, we_ref, fb_ref, g1_ref, b1_ref,
             out_ref):
        p = jnp.dot(af_ref[...], ws_ref[...],
                    preferred_element_type=jnp.float32) + fb_ref[...]
        z = (z_ref[...].astype(jnp.float32)
             + jnp.dot(f_ref[...], we_ref[...],
                       preferred_element_type=jnp.float32))
        zsum = z.reshape(_BN_NODES, _M, 2 * _AF).sum(axis=1)
        s1 = z.sum(axis=0, keepdims=True)
        s2 = (z * z).sum(axis=0, keepdims=True)
        t1 = (p * zsum).sum(axis=0, keepdims=True)
        p1 = p.sum(axis=0, keepdims=True)
        p2 = (p * p).sum(axis=0, keepdims=True)
        blk = jnp.concatenate([s1, s2, t1, p1, p2,
                               jnp.zeros((3, 2 * _AF), jnp.float32)], axis=0)

        @pl.when(pl.program_id(0) == 0)
        def _():
            out_ref[0] = blk

        @pl.when(pl.program_id(0) != 0)
        def _():
            out_ref[0] += blk

        @pl.when(pl.program_id(0) == _CTILES - 1)
        def _():
            acc = out_ref[0]
            colsum = acc[0:1] + float(_M) * acc[3:4]
            colsq = acc[1:2] + 2.0 * acc[2:3] + float(_M) * acc[4:5]
            mu = colsum / nm
            var = colsq / nm - mu * mu
            inv = g1_ref[...] * jax.lax.rsqrt(var + _EPS)
            sh = b1_ref[...] - mu * inv
            out_ref[1] = jnp.concatenate(
                [inv, sh, jnp.zeros((6, 2 * _AF), jnp.float32)], axis=0)

    return pl.pallas_call(
        body,
        grid=(_CTILES,),
        in_specs=[
            pl.BlockSpec((_BN_EDGES, 2 * _AF), lambda i: (i, 0)),
            pl.BlockSpec((_BN_EDGES, _NBR), lambda i: (i + toff, 0)),
            pl.BlockSpec((_BN_NODES, 128), lambda i: (i + toff, 0)),
            pl.BlockSpec((128, 2 * _AF), lambda i: (0, 0)),
            pl.BlockSpec((_NBR, 2 * _AF), lambda i: (0, 0)),
            pl.BlockSpec((1, 2 * _AF), lambda i: (0, 0)),
            pl.BlockSpec((1, 2 * _AF), lambda i: (0, 0)),
            pl.BlockSpec((1, 2 * _AF), lambda i: (0, 0)),
        ],
        out_specs=pl.BlockSpec((2, 8, 2 * _AF), lambda i: (0, 0, 0)),
        out_shape=jax.ShapeDtypeStruct((2, 8, 2 * _AF), jnp.float32),
    )(Z, F, af_pad, ws_t, we_t, fb_row, g1_row, b1_row)


def _conv_apply(Z, F, af_pad, ws_t, we_t, fb_row, st, g2_row, b2_row, toff):
    nn = float(_N)

    def body(z_ref, f_ref, af_ref, ws_ref, we_ref, fb_ref, st_ref,
             g2_ref, b2_ref, s_ref, q_ref):
        p = jnp.dot(af_ref[...], ws_ref[...],
                    preferred_element_type=jnp.float32) + fb_ref[...]
        z = (z_ref[...].astype(jnp.float32)
             + jnp.dot(f_ref[...], we_ref[...],
                       preferred_element_type=jnp.float32))
        sc1 = st_ref[0, 0:1, :]
        sh1 = st_ref[0, 1:2, :]
        gated = z.reshape(_BN_NODES, _M, 2 * _AF) + p[:, None, :]
        gated = gated * sc1[None, :, :] + sh1[None, :, :]
        filt = gated[:, :, :_AF]
        core = gated[:, :, _AF:]
        y = _sigmoid(filt) * _softplus(core)
        s = y.sum(axis=1)
        s_pad = jnp.concatenate(
            [s, jnp.zeros((_BN_NODES, _AF), jnp.float32)], axis=1)
        s_ref[...] = s_pad
        q1 = (s_pad.sum(axis=0, keepdims=True))
        q2 = (s_pad * s_pad).sum(axis=0, keepdims=True)
        blk = jnp.concatenate([q1, q2], axis=0)

        @pl.when(pl.program_id(0) == 0)
        def _():
            q_ref[0] = blk

        @pl.when(pl.program_id(0) != 0)
        def _():
            q_ref[0] += blk

        @pl.when(pl.program_id(0) == _CTILES - 1)
        def _():
            acc = q_ref[0]
            mu2 = acc[0:1] / nn
            var2 = acc[1:2] / nn - mu2 * mu2
            inv2 = g2_ref[...] * jax.lax.rsqrt(var2 + _EPS)
            sh2 = b2_ref[...] - mu2 * inv2
            q_ref[1] = jnp.concatenate([inv2, sh2], axis=0)

    return pl.pallas_call(
        body,
        grid=(_CTILES,),
        in_specs=[
            pl.BlockSpec((_BN_EDGES, 2 * _AF), lambda i: (i, 0)),
            pl.BlockSpec((_BN_EDGES, _NBR), lambda i: (i + toff, 0)),
            pl.BlockSpec((_BN_NODES, 128), lambda i: (i + toff, 0)),
            pl.BlockSpec((128, 2 * _AF), lambda i: (0, 0)),
            pl.BlockSpec((_NBR, 2 * _AF), lambda i: (0, 0)),
            pl.BlockSpec((1, 2 * _AF), lambda i: (0, 0)),
            pl.BlockSpec((1, 8, 2 * _AF), lambda i: (1, 0, 0)),
            pl.BlockSpec((1, 2 * _AF), lambda i: (0, 0)),
            pl.BlockSpec((1, 2 * _AF), lambda i: (0, 0)),
        ],
        out_specs=[
            pl.BlockSpec((_BN_NODES, 128), lambda i: (i, 0)),
            pl.BlockSpec((2, 2, 2 * _AF), lambda i: (0, 0, 0)),
        ],
        out_shape=[
            jax.ShapeDtypeStruct((_CN, 128), jnp.float32),
            jax.ShapeDtypeStruct((2, 2, 2 * _AF), jnp.float32),
        ],
    )(Z, F, af_pad, ws_t, we_t, fb_row, st, g2_row, b2_row)


def _conv_finalize(af_pad, s_pad, q, lanemask, wnext_t=None):
    Bn = 2000
    grid = _N // Bn

    if wnext_t is None:
        def body(af_ref, s_ref, q_ref, lm_ref, out_ref):
            sc2 = q_ref[0, 0:1, :]
            sh2 = q_ref[0, 1:2, :]
            out_ref[...] = _softplus(
                af_ref[...] + s_ref[...] * sc2 + sh2) * lm_ref[...]

        return pl.pallas_call(
            body,
            grid=(grid,),
            in_specs=[
                pl.BlockSpec((Bn, 128), lambda i: (i, 0)),
                pl.BlockSpec((Bn, 128), lambda i: (i, 0)),
                pl.BlockSpec((1, 2, 128), lambda i: (1, 0, 0)),
                pl.BlockSpec((1, 128), lambda i: (0, 0)),
            ],
            out_specs=pl.BlockSpec((Bn, 128), lambda i: (i, 0)),
            out_shape=jax.ShapeDtypeStruct((_N, 128), jnp.float32),
        )(af_pad, s_pad, q, lanemask)

    def body(af_ref, s_ref, q_ref, lm_ref, wn_ref, out_ref, p_ref):
        sc2 = q_ref[0, 0:1, :]
        sh2 = q_ref[0, 1:2, :]
        af_new = _softplus(
            af_ref[...] + s_ref[...] * sc2 + sh2) * lm_ref[...]
        out_ref[...] = af_new
        p_ref[...] = jnp.dot(af_new, wn_ref[...],
                             preferred_element_type=jnp.float32)

    return pl.pallas_call(
        body,
        grid=(grid,),
        in_specs=[
            pl.BlockSpec((Bn, 128), lambda i: (i, 0)),
            pl.BlockSpec((Bn, 128), lambda i: (i, 0)),
            pl.BlockSpec((1, 2, 128), lambda i: (1, 0, 0)),
            pl.BlockSpec((1, 128), lambda i: (0, 0)),
            pl.BlockSpec((128, 128), lambda i: (0, 0)),
        ],
        out_specs=[
            pl.BlockSpec((Bn, 128), lambda i: (i, 0)),
            pl.BlockSpec((Bn, 128), lambda i: (i, 0)),
        ],
        out_shape=[
            jax.ShapeDtypeStruct((_N, 128), jnp.float32),
            jax.ShapeDtypeStruct((_N, 128), jnp.float32),
        ],
    )(af_pad, s_pad, q, lanemask, wnext_t)


def _readout(rows, fc1_wt, fc1_b, fc2_wt, fc2_b, out_wt, out_b, ncry, apc):
    tot = ncry * apc

    def body(r_ref, w1_ref, b1_ref, w2_ref, b2_ref, wo_ref, bo_ref, o_ref):
        r = r_ref[...]
        nrm = jnp.sqrt((r * r).sum(axis=1, keepdims=True))
        g = r / jnp.maximum(nrm, 1e-12)
        pooled = g.reshape(ncry, apc, 128).mean(axis=1)
        h = _softplus(jnp.dot(pooled, w1_ref[...],
                              preferred_element_type=jnp.float32) + b1_ref[...])
        h = _softplus(jnp.dot(h, w2_ref[...],
                              preferred_element_type=jnp.float32) + b2_ref[...])
        props = (jnp.dot(h, wo_ref[...], preferred_element_type=jnp.float32)
                 + bo_ref[...])
        o_ref[...] = props

    return pl.pallas_call(
        body,
        grid=(1,),
        in_specs=[
            pl.BlockSpec((tot, 128), lambda i: (0, 0)),
            pl.BlockSpec((128, _AF), lambda i: (0, 0)),
            pl.BlockSpec((1, _AF), lambda i: (0, 0)),
            pl.BlockSpec((_AF, _AF), lambda i: (0, 0)),
            pl.BlockSpec((1, _AF), lambda i: (0, 0)),
            pl.BlockSpec((_AF, 1), lambda i: (0, 0)),
            pl.BlockSpec((1, 1), lambda i: (0, 0)),
        ],
        out_specs=pl.BlockSpec((ncry, 1), lambda i: (0, 0)),
        out_shape=jax.ShapeDtypeStruct((ncry, 1), jnp.float32),
    )(rows, fc1_wt, fc1_b, fc2_wt, fc2_b, out_wt, out_b)


def kernel(atom_fea, nbr_fea, nbr_fea_idx, crystal_atom_idx, mask, w_emb,
           conv0_fc_w, conv0_fc_b, conv0_bn1_g, conv0_bn1_b, conv0_bn2_g,
           conv0_bn2_b, conv1_fc_w, conv1_fc_b, conv1_bn1_g, conv1_bn1_b,
           conv1_bn2_g, conv1_bn2_b, conv2_fc_w, conv2_fc_b, conv2_bn1_g,
           conv2_bn1_b, conv2_bn2_g, conv2_bn2_b, fc1_w, fc1_b, fc2_w, fc2_b,
           out_w, out_b):
    f32 = jnp.float32
    zpad64 = jnp.zeros((_AF, 2 * _AF), f32)

    def _wsplit(fw):
        fwt = fw.T
        ws_t = jnp.concatenate([fwt[:_AF], zpad64], axis=0)
        wn_t = jnp.concatenate([fwt[_AF:2 * _AF], zpad64], axis=0)
        we_t = fwt[2 * _AF:]
        return ws_t, wn_t, we_t

    wsplits = [_wsplit(conv0_fc_w), _wsplit(conv1_fc_w), _wsplit(conv2_fc_w)]

    mask_row = mask.reshape(1, _ORIG)
    w_emb_t = jnp.concatenate(
        [w_emb.T, jnp.zeros((_ORIG, 128 - _AF), f32)], axis=1)
    masked, af, P_nbr = _embed(atom_fea, mask_row, w_emb_t, wsplits[0][1])

    idx2d = nbr_fea_idx.reshape(_NSTR, 128).astype(jnp.int32)
    F = nbr_fea.reshape(_E, _NBR)
    lanemask = jnp.concatenate(
        [jnp.ones((1, _AF), f32), jnp.zeros((1, _AF), f32)], axis=1)

    convp = [(conv0_fc_b, conv0_bn1_g, conv0_bn1_b, conv0_bn2_g, conv0_bn2_b),
             (conv1_fc_b, conv1_bn1_g, conv1_bn1_b, conv1_bn2_g, conv1_bn2_b),
             (conv2_fc_b, conv2_bn1_g, conv2_bn1_b, conv2_bn2_g, conv2_bn2_b)]

    zrow = jnp.zeros((1, _AF), f32)
    for li, (fb, g1, b1, g2, b2) in enumerate(convp):
        ws_t, _, we_t = wsplits[li]
        fb_row = fb.reshape(1, 2 * _AF)
        g1_row = g1.reshape(1, 2 * _AF)
        b1_row = b1.reshape(1, 2 * _AF)
        g2_row = jnp.concatenate([g2.reshape(1, _AF), zrow], axis=1)
        b2_row = jnp.concatenate([b2.reshape(1, _AF), zrow], axis=1)

        Z = _sc_gather(P_nbr, idx2d, 0, _NSTR)
        st = _conv_stats(Z, F, af, ws_t, we_t, fb_row, g1_row, b1_row, 0)
        s_pad, q = _conv_apply(Z, F, af, ws_t, we_t, fb_row, st,
                               g2_row, b2_row, 0)

        if li < 2:
            af, P_nbr = _conv_finalize(af, s_pad, q, lanemask,
                                       wsplits[li + 1][1])
        else:
            af = _conv_finalize(af, s_pad, q, lanemask)

    ncry, apc = crystal_atom_idx.shape
    cidx = crystal_atom_idx.reshape((ncry * apc) // 128, 128).astype(jnp.int32)
    rows = _sc_gather(af, cidx, 0, (ncry * apc) // 128)

    fc1_wt = jnp.concatenate([fc1_w.T, jnp.zeros((_AF, _AF), f32)], axis=0)
    props = _readout(rows, fc1_wt, fc1_b.reshape(1, _AF), fc2_w.T,
                     fc2_b.reshape(1, _AF), out_w.T, out_b.reshape(1, 1),
                     ncry, apc)
    return props, masked


# speedup vs baseline: 1.1702x; 1.0394x over previous
import functools

import jax
import jax.numpy as jnp
from jax import lax
from jax.experimental import pallas as pl
from jax.experimental.pallas import tpu as pltpu
from jax.experimental.pallas import tpu_sc as plsc

_N = 50000
_M = 16
_ORIG = 128
_NBR = 16
_AF = 64
_E = _N * _M

_NC = 2
_NS = 16
_NW = _NC * _NS

_EPS = 1e-5

_NSTR = _E // 128
_CHUNKS = 1
_CSTR = _NSTR // _CHUNKS
_CE = _CSTR * 128
_CN = _CE // _M

_BN_NODES = 1000
_BN_EDGES = _BN_NODES * _M
_CTILES = _CN // _BN_NODES


def _sigmoid(x):
    return 1.0 / (1.0 + jnp.exp(-x))


def _softplus(x):
    return jnp.maximum(x, 0.0) + jnp.log1p(jnp.exp(-jnp.abs(x)))


def _sc_gather(table, idx2d, base, nstr):
    D = table.shape[1]
    kmax = -(-nstr // _NW)
    mesh = plsc.VectorSubcoreMesh(core_axis_name="c", subcore_axis_name="s")

    @functools.partial(
        pl.kernel,
        out_type=jax.ShapeDtypeStruct((nstr * 128, D), table.dtype),
        mesh=mesh,
        scratch_types=[
            pltpu.VMEM((8, 128), jnp.int32),
            pltpu.VMEM((128, D), table.dtype),
            pltpu.VMEM((128, D), table.dtype),
            pltpu.VMEM((128, D), table.dtype),
            pltpu.VMEM((128, D), table.dtype),
            pltpu.SemaphoreType.DMA,
            pltpu.SemaphoreType.DMA,
            pltpu.SemaphoreType.DMA,
            pltpu.SemaphoreType.DMA,
            pltpu.SemaphoreType.DMA,
            pltpu.SemaphoreType.DMA,
            pltpu.SemaphoreType.DMA,
            pltpu.SemaphoreType.DMA,
            pltpu.SemaphoreType.DMA,
            pltpu.SemaphoreType.DMA,
            pltpu.SemaphoreType.DMA,
            pltpu.SemaphoreType.DMA,
        ],
    )
    def gk(table_hbm, idx_hbm, out_hbm, idxv, rows0, rows1, rows2, rows3,
           is0, is1, is2, is3, is4, is5, is6, is7,
           gs0, gs1, gs2, gs3):
        w = lax.axis_index("s") * _NC + lax.axis_index("c")
        isems = (is0, is1, is2, is3, is4, is5, is6, is7)
        rows = (rows0, rows1, rows2, rows3)
        gsems = (gs0, gs1, gs2, gs3)

        def icopy(k, j):
            g = w + k * _NW

            @pl.when(g < nstr)
            def _():
                pltpu.async_copy(idx_hbm.at[base + g], idxv.at[j], isems[j])

        def iwait(k, j):
            g = w + k * _NW

            @pl.when(g < nstr)
            def _():
                pltpu.make_async_copy(
                    idx_hbm.at[base + g], idxv.at[j], isems[j]).wait()

        def gstart(k, j, s):
            g = w + k * _NW

            @pl.when(g < nstr)
            def _():
                pltpu.async_copy(table_hbm.at[idxv.at[j]], rows[s], gsems[s])

        def gwait_store(k, j, s):
            g = w + k * _NW

            @pl.when(g < nstr)
            def _():
                pltpu.make_async_copy(
                    table_hbm.at[idxv.at[j]], rows[s], gsems[s]).wait()
                pltpu.sync_copy(rows[s], out_hbm.at[pl.ds(g * 128, 128)])

        for j in range(8):
            icopy(j, j)
        for j in range(4):
            iwait(j, j)
            gstart(j, j, j)

        def body(h, carry):
            k0 = 8 * h
            for j in range(8):
                k = k0 + j
                s = j % 4
                gwait_store(k, j, s)
                icopy(k + 8, j)
                iwait(k + 4, (j + 4) % 8)
                gstart(k + 4, (j + 4) % 8, s)
            return carry

        lax.fori_loop(0, -(-kmax // 8), body, 0)

    return gk(table, idx2d)


def _embed(atom_fea, mask_row, w_emb_t, wn_t):
    Bn = 2000
    grid = _N // Bn

    def body(a_ref, m_ref, w_ref, wn_ref, masked_ref, af_ref, p_ref):
        masked = a_ref[...] * m_ref[...]
        masked_ref[...] = masked
        af = jnp.dot(masked, w_ref[...], preferred_element_type=jnp.float32)
        af_ref[...] = af
        p_ref[...] = jnp.dot(af, wn_ref[...],
                             preferred_element_type=jnp.float32)

    return pl.pallas_call(
        body,
        grid=(grid,),
        in_specs=[
            pl.BlockSpec((Bn, _ORIG), lambda i: (i, 0)),
            pl.BlockSpec((1, _ORIG), lambda i: (0, 0)),
            pl.BlockSpec((_ORIG, 128), lambda i: (0, 0)),
            pl.BlockSpec((128, 128), lambda i: (0, 0)),
        ],
        out_specs=[
            pl.BlockSpec((Bn, _ORIG), lambda i: (i, 0)),
            pl.BlockSpec((Bn, 128), lambda i: (i, 0)),
            pl.BlockSpec((Bn, 128), lambda i: (i, 0)),
        ],
        out_shape=[
            jax.ShapeDtypeStruct((_N, _ORIG), jnp.float32),
            jax.ShapeDtypeStruct((_N, 128), jnp.float32),
            jax.ShapeDtypeStruct((_N, 128), jnp.float32),
        ],
    )(atom_fea, mask_row, w_emb_t, wn_t)


def _conv_stats(Z, F, af_pad, ws_t, we_t, fb_row, g1_row, b1_row, toff):
    nm = float(_E)

    def body(z_ref, f_ref, af_ref, ws_ref, we_ref, fb_ref, g1_ref, b1_ref,
             out_ref):
        p = jnp.dot(af_ref[...], ws_ref[...],
                    preferred_element_type=jnp.float32) + fb_ref[...]
        z = (z_ref[...].astype(jnp.float32)
             + jnp.dot(f_ref[...], we_ref[...],
                       preferred_element_type=jnp.float32))
        zsum = z.reshape(_BN_NODES, _M, 2 * _AF).sum(axis=1)
        s1 = z.sum(axis=0, keepdims=True)
        s2 = (z * z).sum(axis=0, keepdims=True)
        t1 = (p * zsum).sum(axis=0, keepdims=True)
        p1 = p.sum(axis=0, keepdims=True)
        p2 = (p * p).sum(axis=0, keepdims=True)
        blk = jnp.concatenate([s1, s2, t1, p1, p2,
                               jnp.zeros((3, 2 * _AF), jnp.float32)], axis=0)

        @pl.when(pl.program_id(0) == 0)
        def _():
            out_ref[0] = blk

        @pl.when(pl.program_id(0) != 0)
        def _():
            out_ref[0] += blk

        @pl.when(pl.program_id(0) == _CTILES - 1)
        def _():
            acc = out_ref[0]
            colsum = acc[0:1] + float(_M) * acc[3:4]
            colsq = acc[1:2] + 2.0 * acc[2:3] + float(_M) * acc[4:5]
            mu = colsum / nm
            var = colsq / nm - mu * mu
            inv = g1_ref[...] * jax.lax.rsqrt(var + _EPS)
            sh = b1_ref[...] - mu * inv
            out_ref[1] = jnp.concatenate(
                [inv, sh, jnp.zeros((6, 2 * _AF), jnp.float32)], axis=0)

    return pl.pallas_call(
        body,
        grid=(_CTILES,),
        in_specs=[
            pl.BlockSpec((_BN_EDGES, 2 * _AF), lambda i: (i, 0)),
            pl.BlockSpec((_BN_EDGES, _NBR), lambda i: (i + toff, 0)),
            pl.BlockSpec((_BN_NODES, 128), lambda i: (i + toff, 0)),
            pl.BlockSpec((128, 2 * _AF), lambda i: (0, 0)),
            pl.BlockSpec((_NBR, 2 * _AF), lambda i: (0, 0)),
            pl.BlockSpec((1, 2 * _AF), lambda i: (0, 0)),
            pl.BlockSpec((1, 2 * _AF), lambda i: (0, 0)),
            pl.BlockSpec((1, 2 * _AF), lambda i: (0, 0)),
        ],
        out_specs=pl.BlockSpec((2, 8, 2 * _AF), lambda i: (0, 0, 0)),
        out_shape=jax.ShapeDtypeStruct((2, 8, 2 * _AF), jnp.float32),
    )(Z, F, af_pad, ws_t, we_t, fb_row, g1_row, b1_row)


def _conv_apply(Z, F, af_pad, ws_t, we_t, fb_row, st, g2_row, b2_row, toff):
    nn = float(_N)

    def body(z_ref, f_ref, af_ref, ws_ref, we_ref, fb_ref, st_ref,
             g2_ref, b2_ref, s_ref, q_ref):
        p = jnp.dot(af_ref[...], ws_ref[...],
                    preferred_element_type=jnp.float32) + fb_ref[...]
        z = (z_ref[...].astype(jnp.float32)
             + jnp.dot(f_ref[...], we_ref[...],
                       preferred_element_type=jnp.float32))
        sc1 = st_ref[0, 0:1, :]
        sh1 = st_ref[0, 1:2, :]
        gated = z.reshape(_BN_NODES, _M, 2 * _AF) + p[:, None, :]
        gated = gated * sc1[None, :, :] + sh1[None, :, :]
        filt = gated[:, :, :_AF]
        core = gated[:, :, _AF:]
        y = _sigmoid(filt) * _softplus(core)
        s = y.sum(axis=1)
        s_pad = jnp.concatenate(
            [s, jnp.zeros((_BN_NODES, _AF), jnp.float32)], axis=1)
        s_ref[...] = s_pad
        q1 = (s_pad.sum(axis=0, keepdims=True))
        q2 = (s_pad * s_pad).sum(axis=0, keepdims=True)
        blk = jnp.concatenate([q1, q2], axis=0)

        @pl.when(pl.program_id(0) == 0)
        def _():
            q_ref[0] = blk

        @pl.when(pl.program_id(0) != 0)
        def _():
            q_ref[0] += blk

        @pl.when(pl.program_id(0) == _CTILES - 1)
        def _():
            acc = q_ref[0]
            mu2 = acc[0:1] / nn
            var2 = acc[1:2] / nn - mu2 * mu2
            inv2 = g2_ref[...] * jax.lax.rsqrt(var2 + _EPS)
            sh2 = b2_ref[...] - mu2 * inv2
            q_ref[1] = jnp.concatenate([inv2, sh2], axis=0)

    return pl.pallas_call(
        body,
        grid=(_CTILES,),
        in_specs=[
            pl.BlockSpec((_BN_EDGES, 2 * _AF), lambda i: (i, 0)),
            pl.BlockSpec((_BN_EDGES, _NBR), lambda i: (i + toff, 0)),
            pl.BlockSpec((_BN_NODES, 128), lambda i: (i + toff, 0)),
            pl.BlockSpec((128, 2 * _AF), lambda i: (0, 0)),
            pl.BlockSpec((_NBR, 2 * _AF), lambda i: (0, 0)),
            pl.BlockSpec((1, 2 * _AF), lambda i: (0, 0)),
            pl.BlockSpec((1, 8, 2 * _AF), lambda i: (1, 0, 0)),
            pl.BlockSpec((1, 2 * _AF), lambda i: (0, 0)),
            pl.BlockSpec((1, 2 * _AF), lambda i: (0, 0)),
        ],
        out_specs=[
            pl.BlockSpec((_BN_NODES, 128), lambda i: (i, 0)),
            pl.BlockSpec((2, 2, 2 * _AF), lambda i: (0, 0, 0)),
        ],
        out_shape=[
            jax.ShapeDtypeStruct((_CN, 128), jnp.float32),
            jax.ShapeDtypeStruct((2, 2, 2 * _AF), jnp.float32),
        ],
    )(Z, F, af_pad, ws_t, we_t, fb_row, st, g2_row, b2_row)


def _conv_finalize(af_pad, s_pad, q, lanemask, wnext_t=None):
    Bn = 2000
    grid = _N // Bn

    if wnext_t is None:
        def body(af_ref, s_ref, q_ref, lm_ref, out_ref):
            sc2 = q_ref[0, 0:1, :]
            sh2 = q_ref[0, 1:2, :]
            out_ref[...] = _softplus(
                af_ref[...] + s_ref[...] * sc2 + sh2) * lm_ref[...]

        return pl.pallas_call(
            body,
            grid=(grid,),
            in_specs=[
                pl.BlockSpec((Bn, 128), lambda i: (i, 0)),
                pl.BlockSpec((Bn, 128), lambda i: (i, 0)),
                pl.BlockSpec((1, 2, 128), lambda i: (1, 0, 0)),
                pl.BlockSpec((1, 128), lambda i: (0, 0)),
            ],
            out_specs=pl.BlockSpec((Bn, 128), lambda i: (i, 0)),
            out_shape=jax.ShapeDtypeStruct((_N, 128), jnp.float32),
        )(af_pad, s_pad, q, lanemask)

    def body(af_ref, s_ref, q_ref, lm_ref, wn_ref, out_ref, p_ref):
        sc2 = q_ref[0, 0:1, :]
        sh2 = q_ref[0, 1:2, :]
        af_new = _softplus(
            af_ref[...] + s_ref[...] * sc2 + sh2) * lm_ref[...]
        out_ref[...] = af_new
        p_ref[...] = jnp.dot(af_new, wn_ref[...],
                             preferred_element_type=jnp.float32)

    return pl.pallas_call(
        body,
        grid=(grid,),
        in_specs=[
            pl.BlockSpec((Bn, 128), lambda i: (i, 0)),
            pl.BlockSpec((Bn, 128), lambda i: (i, 0)),
            pl.BlockSpec((1, 2, 128), lambda i: (1, 0, 0)),
            pl.BlockSpec((1, 128), lambda i: (0, 0)),
            pl.BlockSpec((128, 128), lambda i: (0, 0)),
        ],
        out_specs=[
            pl.BlockSpec((Bn, 128), lambda i: (i, 0)),
            pl.BlockSpec((Bn, 128), lambda i: (i, 0)),
        ],
        out_shape=[
            jax.ShapeDtypeStruct((_N, 128), jnp.float32),
            jax.ShapeDtypeStruct((_N, 128), jnp.float32),
        ],
    )(af_pad, s_pad, q, lanemask, wnext_t)


def _readout(rows, fc1_wt, fc1_b, fc2_wt, fc2_b, out_wt, out_b, ncry, apc):
    tot = ncry * apc

    def body(r_ref, w1_ref, b1_ref, w2_ref, b2_ref, wo_ref, bo_ref, o_ref):
        r = r_ref[...]
        nrm = jnp.sqrt((r * r).sum(axis=1, keepdims=True))
        g = r / jnp.maximum(nrm, 1e-12)
        pooled = g.reshape(ncry, apc, 128).mean(axis=1)
        h = _softplus(jnp.dot(pooled, w1_ref[...],
                              preferred_element_type=jnp.float32) + b1_ref[...])
        h = _softplus(jnp.dot(h, w2_ref[...],
                              preferred_element_type=jnp.float32) + b2_ref[...])
        props = (jnp.dot(h, wo_ref[...], preferred_element_type=jnp.float32)
                 + bo_ref[...])
        o_ref[...] = props

    return pl.pallas_call(
        body,
        grid=(1,),
        in_specs=[
            pl.BlockSpec((tot, 128), lambda i: (0, 0)),
            pl.BlockSpec((128, _AF), lambda i: (0, 0)),
            pl.BlockSpec((1, _AF), lambda i: (0, 0)),
            pl.BlockSpec((_AF, _AF), lambda i: (0, 0)),
            pl.BlockSpec((1, _AF), lambda i: (0, 0)),
            pl.BlockSpec((_AF, 1), lambda i: (0, 0)),
            pl.BlockSpec((1, 1), lambda i: (0, 0)),
        ],
        out_specs=pl.BlockSpec((ncry, 1), lambda i: (0, 0)),
        out_shape=jax.ShapeDtypeStruct((ncry, 1), jnp.float32),
    )(rows, fc1_wt, fc1_b, fc2_wt, fc2_b, out_wt, out_b)


def kernel(atom_fea, nbr_fea, nbr_fea_idx, crystal_atom_idx, mask, w_emb,
           conv0_fc_w, conv0_fc_b, conv0_bn1_g, conv0_bn1_b, conv0_bn2_g,
           conv0_bn2_b, conv1_fc_w, conv1_fc_b, conv1_bn1_g, conv1_bn1_b,
           conv1_bn2_g, conv1_bn2_b, conv2_fc_w, conv2_fc_b, conv2_bn1_g,
           conv2_bn1_b, conv2_bn2_g, conv2_bn2_b, fc1_w, fc1_b, fc2_w, fc2_b,
           out_w, out_b):
    f32 = jnp.float32
    zpad64 = jnp.zeros((_AF, 2 * _AF), f32)

    def _wsplit(fw):
        fwt = fw.T
        ws_t = jnp.concatenate([fwt[:_AF], zpad64], axis=0)
        wn_t = jnp.concatenate([fwt[_AF:2 * _AF], zpad64], axis=0)
        we_t = fwt[2 * _AF:]
        return ws_t, wn_t, we_t

    wsplits = [_wsplit(conv0_fc_w), _wsplit(conv1_fc_w), _wsplit(conv2_fc_w)]

    mask_row = mask.reshape(1, _ORIG)
    w_emb_t = jnp.concatenate(
        [w_emb.T, jnp.zeros((_ORIG, 128 - _AF), f32)], axis=1)
    masked, af, P_nbr = _embed(atom_fea, mask_row, w_emb_t, wsplits[0][1])

    idx2d = nbr_fea_idx.reshape(_NSTR, 128).astype(jnp.int32)
    F = nbr_fea.reshape(_E, _NBR)
    lanemask = jnp.concatenate(
        [jnp.ones((1, _AF), f32), jnp.zeros((1, _AF), f32)], axis=1)

    convp = [(conv0_fc_b, conv0_bn1_g, conv0_bn1_b, conv0_bn2_g, conv0_bn2_b),
             (conv1_fc_b, conv1_bn1_g, conv1_bn1_b, conv1_bn2_g, conv1_bn2_b),
             (conv2_fc_b, conv2_bn1_g, conv2_bn1_b, conv2_bn2_g, conv2_bn2_b)]

    zrow = jnp.zeros((1, _AF), f32)
    for li, (fb, g1, b1, g2, b2) in enumerate(convp):
        ws_t, _, we_t = wsplits[li]
        fb_row = fb.reshape(1, 2 * _AF)
        g1_row = g1.reshape(1, 2 * _AF)
        b1_row = b1.reshape(1, 2 * _AF)
        g2_row = jnp.concatenate([g2.reshape(1, _AF), zrow], axis=1)
        b2_row = jnp.concatenate([b2.reshape(1, _AF), zrow], axis=1)

        Z = _sc_gather(P_nbr, idx2d, 0, _NSTR)
        st = _conv_stats(Z, F, af, ws_t, we_t, fb_row, g1_row, b1_row, 0)
        s_pad, q = _conv_apply(Z, F, af, ws_t, we_t, fb_row, st,
                               g2_row, b2_row, 0)

        if li < 2:
            af, P_nbr = _conv_finalize(af, s_pad, q, lanemask,
                                       wsplits[li + 1][1])
        else:
            af = _conv_finalize(af, s_pad, q, lanemask)

    ncry, apc = crystal_atom_idx.shape
    cidx = crystal_atom_idx.reshape((ncry * apc) // 128, 128).astype(jnp.int32)
    rows = _sc_gather(af, cidx, 0, (ncry * apc) // 128)

    fc1_wt = jnp.concatenate([fc1_w.T, jnp.zeros((_AF, _AF), f32)], axis=0)
    props = _readout(rows, fc1_wt, fc1_b.reshape(1, _AF), fc2_w.T,
                     fc2_b.reshape(1, _AF), out_w.T, out_b.reshape(1, 1),
                     ncry, apc)
    return props, masked
